# baseline (speedup 1.0000x reference)
import jax
import jax.numpy as jnp
from jax.experimental import pallas as pl
from jax.experimental.pallas import tpu as pltpu

_N = 1024
_NSUB = 512
_STRIP = 128
_KSEL = 15
_G = 2
_NMS_T = 0.5
_SCORE_T = 0.2
_NEG = float(jnp.finfo(jnp.float32).min)


def _build_adjacency(r, cols, m_ref):
    x1r, y1r = r[0:1, :], r[1:2, :]
    x2r, y2r = r[2:3, :], r[3:4, :]
    sr, labr = r[4:5, :], r[5:6, :]
    arear = (x2r - x1r) * (y2r - y1r)
    iota_v = jax.lax.broadcasted_iota(jnp.int32, (_STRIP, _N), 1)
    iota_u0 = jax.lax.broadcasted_iota(jnp.int32, (_STRIP, _N), 0)
    for s in range(_N // _STRIP):
        c = cols[s * _STRIP:(s + 1) * _STRIP, :]
        x1c, y1c = c[:, 0:1], c[:, 1:2]
        x2c, y2c = c[:, 2:3], c[:, 3:4]
        sc, labc = c[:, 4:5], c[:, 5:6]
        wx = jnp.maximum(jnp.minimum(x2c, x2r) - jnp.maximum(x1c, x1r), 0.0)
        wy = jnp.maximum(jnp.minimum(y2c, y2r) - jnp.maximum(y1c, y1r), 0.0)
        inter = wx * wy
        areac = (x2c - x1c) * (y2c - y1c)
        union = areac + arear - inter
        ovv = jnp.where((inter + inter) > union,
                        jnp.where(labc == labr, 1.0, 0.0), 0.0)
        iota_u = iota_u0 + s * _STRIP
        precv = jnp.where(
            sc > sr, 1.0,
            jnp.where(sc == sr,
                      jnp.where(iota_u < iota_v, 1.0, 0.0), 0.0))
        m1v = ovv * precv
        diagv = jnp.where(iota_u == iota_v, 1.0, 0.0)
        act_c = jnp.where(sc >= _SCORE_T, 1.0, 0.0)
        m2v = ovv - m1v - diagv
        m_ref[s * _STRIP:(s + 1) * _STRIP, :] = (
            m1v * act_c - m2v).astype(jnp.bfloat16)


def _sweep(kept, active_r, m_ref):
    sup_c = jnp.min(m_ref[...] * kept, axis=1, keepdims=True)
    kept_c = jnp.where(sup_c < 0, jnp.bfloat16(0.0), jnp.bfloat16(1.0))
    sup_r = jnp.max(m_ref[...] * kept_c, axis=0, keepdims=True)
    return active_r * jnp.where(sup_r > 0,
                                jnp.bfloat16(0.0), jnp.bfloat16(1.0))


def _select_topk(r, keptb, out_ref):
    x1r, y1r = r[0:1, :], r[1:2, :]
    x2r, y2r = r[2:3, :], r[3:4, :]
    sr, labr = r[4:5, :], r[5:6, :]
    iota_n = jax.lax.broadcasted_iota(jnp.int32, (1, _N), 1)
    is_subj = iota_n < _NSUB
    key_s = jnp.where(keptb & is_subj, sr, _NEG)
    key_o = jnp.where(keptb & (~is_subj), sr, _NEG)

    lane = jax.lax.broadcasted_iota(jnp.int32, (1, 128), 1)
    x1a = jnp.zeros((1, 128), jnp.float32)
    y1a = jnp.zeros((1, 128), jnp.float32)
    x2a = jnp.zeros((1, 128), jnp.float32)
    y2a = jnp.zeros((1, 128), jnp.float32)
    sca = jnp.zeros((1, 128), jnp.float32)
    laba = jnp.zeros((1, 128), jnp.float32)
    vala = jnp.zeros((1, 128), jnp.float32)

    key = key_s
    for t in range(2 * _KSEL):
        if t == _KSEL:
            key = key_o
        m = jnp.max(key, axis=1, keepdims=True)
        idx = jnp.min(jnp.where(key == m, iota_n, _N),
                      axis=1, keepdims=True)
        v = jnp.where(m > _NEG, 1.0, 0.0)
        sel = iota_n == idx
        gx1 = jnp.sum(jnp.where(sel, x1r, 0.0), axis=1, keepdims=True)
        gy1 = jnp.sum(jnp.where(sel, y1r, 0.0), axis=1, keepdims=True)
        gx2 = jnp.sum(jnp.where(sel, x2r, 0.0), axis=1, keepdims=True)
        gy2 = jnp.sum(jnp.where(sel, y2r, 0.0), axis=1, keepdims=True)
        glab = jnp.sum(jnp.where(sel, labr, 0.0), axis=1, keepdims=True)
        ot = lane == t
        x1a = x1a + jnp.where(ot, gx1 * v, 0.0)
        y1a = y1a + jnp.where(ot, gy1 * v, 0.0)
        x2a = x2a + jnp.where(ot, gx2 * v, 0.0)
        y2a = y2a + jnp.where(ot, gy2 * v, 0.0)
        sca = sca + jnp.where(ot, m * v, 0.0)
        laba = laba + jnp.where(ot, glab * v - (1.0 - v), 0.0)
        vala = vala + jnp.where(ot, v, 0.0)
        key = jnp.where(sel, _NEG, key)

    ns = jnp.sum(jnp.where(lane < _KSEL, vala, 0.0), axis=1, keepdims=True)
    ns_row = jnp.broadcast_to(ns, (1, 128))
    out_ref[...] = jnp.concatenate(
        [x1a, y1a, x2a, y2a, sca, laba, vala, ns_row], axis=0)


def _nms_kernel(rows_ref, out_ref, m_ref, cols_ref):
    rs = [rows_ref[g] for g in range(_G)]
    for g in range(_G):
        cols_ref[g] = jnp.transpose(rs[g][0:6, :], (1, 0))
    for g in range(_G):
        _build_adjacency(rs[g], cols_ref[g], m_ref.at[g])

    active = [jnp.where(rs[g][4:5, :] >= _SCORE_T, 1.0,
                        0.0).astype(jnp.bfloat16) for g in range(_G)]

    k1 = [_sweep(active[g], active[g], m_ref.at[g]) for g in range(_G)]
    k2 = [_sweep(k1[g], active[g], m_ref.at[g]) for g in range(_G)]
    k3 = [_sweep(k2[g], active[g], m_ref.at[g]) for g in range(_G)]
    kept0 = jnp.concatenate(k3, axis=0)
    diff0 = (kept0 - jnp.concatenate(k2, axis=0)).astype(jnp.float32)

    def w_cond(carry):
        return carry[1]

    def w_body(carry):
        kept, _ = carry
        new = [_sweep(kept[g:g + 1, :], active[g], m_ref.at[g])
               for g in range(_G)]
        new = jnp.concatenate(new, axis=0)
        diff = (new - kept).astype(jnp.float32)
        return new, jnp.any(diff != 0.0)

    kept, _ = jax.lax.while_loop(w_cond, w_body,
                                 (kept0, jnp.any(diff0 != 0.0)))

    for g in range(_G):
        keptb = kept[g:g + 1, :].astype(jnp.float32) > 0.5
        _select_topk(rs[g], keptb, out_ref.at[g])


def kernel(subj_boxes, subj_scores, subj_labels, obj_boxes, obj_scores,
           obj_labels):
    b = subj_boxes.shape[0]
    boxes = jnp.concatenate([subj_boxes, obj_boxes], axis=1)
    scores = jnp.concatenate([subj_scores, obj_scores], axis=1)
    labels = jnp.concatenate([subj_labels, obj_labels],
                             axis=1).astype(jnp.float32)
    rows = jnp.concatenate(
        [jnp.swapaxes(boxes, 1, 2), scores[:, None, :],
         labels[:, None, :], jnp.zeros((b, 2, _N), jnp.float32)],
        axis=1)

    out = pl.pallas_call(
        _nms_kernel,
        grid=(b // _G,),
        in_specs=[
            pl.BlockSpec((_G, 8, _N), lambda i: (i, 0, 0)),
        ],
        out_specs=pl.BlockSpec((_G, 8, 128), lambda i: (i, 0, 0)),
        out_shape=jax.ShapeDtypeStruct((b, 8, 128), jnp.float32),
        scratch_shapes=[
            pltpu.VMEM((_G, _N, _N), jnp.bfloat16),
            pltpu.VMEM((_G, _N, 6), jnp.float32),
        ],
        compiler_params=pltpu.CompilerParams(
            dimension_semantics=("parallel",),
            vmem_limit_bytes=48 * 1024 * 1024,
        ),
    )(rows)

    k2 = 2 * _KSEL
    out_boxes = jnp.swapaxes(out[:, 0:4, 0:k2], 1, 2)
    out_scores = out[:, 4, 0:k2]
    out_labels = out[:, 5, 0:k2].astype(jnp.int32)
    valid = out[:, 6, 0:k2] > 0.5
    num_subjects = out[:, 7, 0].astype(jnp.int32)
    return out_boxes, out_scores, out_labels, num_subjects, valid


# speedup vs baseline: 1.0744x; 1.0744x over previous
import jax
import jax.numpy as jnp
from jax.experimental import pallas as pl
from jax.experimental.pallas import tpu as pltpu

_N = 1024
_NSUB = 512
_STRIP = 128
_KSEL = 15
_G = 2
_NMS_T = 0.5
_SCORE_T = 0.2
_NEG = float(jnp.finfo(jnp.float32).min)


def _build_adjacency(r, cols, m1_ref, m2_ref):
    x1r, y1r = r[0:1, :], r[1:2, :]
    x2r, y2r = r[2:3, :], r[3:4, :]
    sr, labr = r[4:5, :], r[5:6, :]
    arear = (x2r - x1r) * (y2r - y1r)
    iota_v = jax.lax.broadcasted_iota(jnp.int32, (_STRIP, _N), 1)
    iota_u0 = jax.lax.broadcasted_iota(jnp.int32, (_STRIP, _N), 0)
    for s in range(_N // _STRIP):
        c = cols[s * _STRIP:(s + 1) * _STRIP, :]
        x1c, y1c = c[:, 0:1], c[:, 1:2]
        x2c, y2c = c[:, 2:3], c[:, 3:4]
        sc, labc = c[:, 4:5], c[:, 5:6]
        wx = jnp.maximum(jnp.minimum(x2c, x2r) - jnp.maximum(x1c, x1r), 0.0)
        wy = jnp.maximum(jnp.minimum(y2c, y2r) - jnp.maximum(y1c, y1r), 0.0)
        inter = wx * wy
        areac = (x2c - x1c) * (y2c - y1c)
        union = areac + arear - inter
        ovv = jnp.where((inter + inter) > union,
                        jnp.where(labc == labr, 1.0, 0.0), 0.0)
        iota_u = iota_u0 + s * _STRIP
        precv = jnp.where(
            sc > sr, 1.0,
            jnp.where(sc == sr,
                      jnp.where(iota_u < iota_v, 1.0, 0.0), 0.0))
        m1v = ovv * precv
        diagv = jnp.where(iota_u == iota_v, 1.0, 0.0)
        act_c = jnp.where(sc >= _SCORE_T, 1.0, 0.0)
        m1_ref[s * _STRIP:(s + 1) * _STRIP, :] = (
            m1v * act_c).astype(jnp.bfloat16)
        m2_ref[s * _STRIP:(s + 1) * _STRIP, :] = (
            ovv - m1v - diagv).astype(jnp.bfloat16)


def _sweep(kept, active_r, m1_ref, m2_ref):
    sup_c = jnp.max(m2_ref[...] * kept, axis=1, keepdims=True)
    kept_c = jnp.where(sup_c > 0, jnp.bfloat16(0.0), jnp.bfloat16(1.0))
    sup_r = jnp.max(m1_ref[...] * kept_c, axis=0, keepdims=True)
    return active_r * jnp.where(sup_r > 0,
                                jnp.bfloat16(0.0), jnp.bfloat16(1.0))


def _select_topk(r, keptb, out_ref):
    x1r, y1r = r[0:1, :], r[1:2, :]
    x2r, y2r = r[2:3, :], r[3:4, :]
    sr, labr = r[4:5, :], r[5:6, :]
    iota_n = jax.lax.broadcasted_iota(jnp.int32, (1, _N), 1)
    is_subj = iota_n < _NSUB
    key_s = jnp.where(keptb & is_subj, sr, _NEG)
    key_o = jnp.where(keptb & (~is_subj), sr, _NEG)

    lane = jax.lax.broadcasted_iota(jnp.int32, (1, 128), 1)
    x1a = jnp.zeros((1, 128), jnp.float32)
    y1a = jnp.zeros((1, 128), jnp.float32)
    x2a = jnp.zeros((1, 128), jnp.float32)
    y2a = jnp.zeros((1, 128), jnp.float32)
    sca = jnp.zeros((1, 128), jnp.float32)
    laba = jnp.zeros((1, 128), jnp.float32)
    vala = jnp.zeros((1, 128), jnp.float32)

    key = key_s
    for t in range(2 * _KSEL):
        if t == _KSEL:
            key = key_o
        m = jnp.max(key, axis=1, keepdims=True)
        idx = jnp.min(jnp.where(key == m, iota_n, _N),
                      axis=1, keepdims=True)
        v = jnp.where(m > _NEG, 1.0, 0.0)
        sel = iota_n == idx
        gx1 = jnp.sum(jnp.where(sel, x1r, 0.0), axis=1, keepdims=True)
        gy1 = jnp.sum(jnp.where(sel, y1r, 0.0), axis=1, keepdims=True)
        gx2 = jnp.sum(jnp.where(sel, x2r, 0.0), axis=1, keepdims=True)
        gy2 = jnp.sum(jnp.where(sel, y2r, 0.0), axis=1, keepdims=True)
        glab = jnp.sum(jnp.where(sel, labr, 0.0), axis=1, keepdims=True)
        ot = lane == t
        x1a = x1a + jnp.where(ot, gx1 * v, 0.0)
        y1a = y1a + jnp.where(ot, gy1 * v, 0.0)
        x2a = x2a + jnp.where(ot, gx2 * v, 0.0)
        y2a = y2a + jnp.where(ot, gy2 * v, 0.0)
        sca = sca + jnp.where(ot, m * v, 0.0)
        laba = laba + jnp.where(ot, glab * v - (1.0 - v), 0.0)
        vala = vala + jnp.where(ot, v, 0.0)
        key = jnp.where(sel, _NEG, key)

    ns = jnp.sum(jnp.where(lane < _KSEL, vala, 0.0), axis=1, keepdims=True)
    ns_row = jnp.broadcast_to(ns, (1, 128))
    out_ref[...] = jnp.concatenate(
        [x1a, y1a, x2a, y2a, sca, laba, vala, ns_row], axis=0)


def _nms_kernel(rows_ref, out_ref, m1_ref, m2_ref, cols_ref):
    rs = [rows_ref[g] for g in range(_G)]
    for g in range(_G):
        cols_ref[g] = jnp.transpose(rs[g][0:6, :], (1, 0))
    for g in range(_G):
        _build_adjacency(rs[g], cols_ref[g], m1_ref.at[g], m2_ref.at[g])

    active = [jnp.where(rs[g][4:5, :] >= _SCORE_T, 1.0,
                        0.0).astype(jnp.bfloat16) for g in range(_G)]

    k1 = [_sweep(active[g], active[g], m1_ref.at[g], m2_ref.at[g])
          for g in range(_G)]
    k2 = [_sweep(k1[g], active[g], m1_ref.at[g], m2_ref.at[g])
          for g in range(_G)]
    kept0 = jnp.concatenate(k2, axis=0)
    diff0 = (kept0 - jnp.concatenate(k1, axis=0)).astype(jnp.float32)

    def w_cond(carry):
        return carry[1]

    def w_body(carry):
        kept, _ = carry
        new = [_sweep(kept[g:g + 1, :], active[g], m1_ref.at[g],
                      m2_ref.at[g]) for g in range(_G)]
        new = jnp.concatenate(new, axis=0)
        diff = (new - kept).astype(jnp.float32)
        return new, jnp.any(diff != 0.0)

    kept, _ = jax.lax.while_loop(w_cond, w_body,
                                 (kept0, jnp.any(diff0 != 0.0)))

    for g in range(_G):
        keptb = kept[g:g + 1, :].astype(jnp.float32) > 0.5
        _select_topk(rs[g], keptb, out_ref.at[g])


def kernel(subj_boxes, subj_scores, subj_labels, obj_boxes, obj_scores,
           obj_labels):
    b = subj_boxes.shape[0]
    boxes = jnp.concatenate([subj_boxes, obj_boxes], axis=1)
    scores = jnp.concatenate([subj_scores, obj_scores], axis=1)
    labels = jnp.concatenate([subj_labels, obj_labels],
                             axis=1).astype(jnp.float32)
    rows = jnp.concatenate(
        [jnp.swapaxes(boxes, 1, 2), scores[:, None, :],
         labels[:, None, :], jnp.zeros((b, 2, _N), jnp.float32)],
        axis=1)

    out = pl.pallas_call(
        _nms_kernel,
        grid=(b // _G,),
        in_specs=[
            pl.BlockSpec((_G, 8, _N), lambda i: (i, 0, 0)),
        ],
        out_specs=pl.BlockSpec((_G, 8, 128), lambda i: (i, 0, 0)),
        out_shape=jax.ShapeDtypeStruct((b, 8, 128), jnp.float32),
        scratch_shapes=[
            pltpu.VMEM((_G, _N, _N), jnp.bfloat16),
            pltpu.VMEM((_G, _N, _N), jnp.bfloat16),
            pltpu.VMEM((_G, _N, 6), jnp.float32),
        ],
        compiler_params=pltpu.CompilerParams(
            dimension_semantics=("parallel",),
            vmem_limit_bytes=48 * 1024 * 1024,
        ),
    )(rows)

    k2 = 2 * _KSEL
    out_boxes = jnp.swapaxes(out[:, 0:4, 0:k2], 1, 2)
    out_scores = out[:, 4, 0:k2]
    out_labels = out[:, 5, 0:k2].astype(jnp.int32)
    valid = out[:, 6, 0:k2] > 0.5
    num_subjects = out[:, 7, 0].astype(jnp.int32)
    return out_boxes, out_scores, out_labels, num_subjects, valid
